# baseline (speedup 1.0000x reference)
import functools

import jax
import jax.numpy as jnp
from jax.experimental import pallas as pl
from jax.experimental.pallas import tpu as pltpu

N_NODES = 50000
E = 800000
ND = 16
NUM_LAYER = 8

BE = 6400
GRID = E // BE


def _mlp_body(we1, be1, we2, be2, wv1, bv1, wv2, bv2, miT, mjT, hT, hout, mout):
    x = jnp.concatenate([miT[...], mjT[...], hT[...]], axis=0)
    t = jax.lax.dot_general(we1[...], x, (((1,), (0,)), ((), ())),
                            preferred_element_type=jnp.float32)
    t = jnp.maximum(t + be1[...], 0.0)
    h = jax.lax.dot_general(we2[...], t, (((1,), (0,)), ((), ())),
                            preferred_element_type=jnp.float32) + be2[...]
    y = jnp.concatenate([miT[...], h], axis=0)
    u = jax.lax.dot_general(wv1[...], y, (((1,), (0,)), ((), ())),
                            preferred_element_type=jnp.float32)
    u = jnp.maximum(u + bv1[...], 0.0)
    m = jax.lax.dot_general(wv2[...], u, (((1,), (0,)), ((), ())),
                            preferred_element_type=jnp.float32) + bv2[...]
    hout[...] = h
    mout[...] = m


def _full(shape):
    return pl.BlockSpec(shape, lambda i: (0,) * len(shape))


_EDGE_SPEC = pl.BlockSpec((ND, BE), lambda i: (0, i))


@jax.jit
def _mlp_layer(we1, be1, we2, be2, wv1, bv1, wv2, bv2, miT, mjT, hT):
    return pl.pallas_call(
        _mlp_body,
        grid=(GRID,),
        in_specs=[
            _full((24, 48)), _full((24, 1)), _full((16, 24)), _full((16, 1)),
            _full((16, 32)), _full((16, 1)), _full((16, 16)), _full((16, 1)),
            _EDGE_SPEC, _EDGE_SPEC, _EDGE_SPEC,
        ],
        out_specs=[_EDGE_SPEC, _EDGE_SPEC],
        out_shape=[
            jax.ShapeDtypeStruct((ND, E), jnp.float32),
            jax.ShapeDtypeStruct((ND, E), jnp.float32),
        ],
    )(we1, be1, we2, be2, wv1, bv1, wv2, bv2, miT, mjT, hT)


def _proj_body(wf, bf, hT, oout):
    o = jax.lax.dot_general(wf[...], hT[...], (((1,), (0,)), ((), ())),
                            preferred_element_type=jnp.float32)
    oout[...] = jnp.maximum(o + bf[...], 0.0)


@jax.jit
def _final_proj(wf, bf, hT):
    return pl.pallas_call(
        _proj_body,
        grid=(GRID,),
        in_specs=[_full((1, 16)), _full((1, 1)), _EDGE_SPEC],
        out_specs=pl.BlockSpec((1, BE), lambda i: (0, i)),
        out_shape=jax.ShapeDtypeStruct((1, E), jnp.float32),
    )(wf, bf, hT)


def kernel(M, H, edge_index, We1, be1, We2, be2, Wv1, bv1, Wv2, bv2, Wf, bf):
    src = edge_index[0]
    dst = edge_index[1]
    perm = jnp.argsort(dst)
    dst_s = jnp.take(dst, perm)
    src_s = jnp.take(src, perm)

    be1c = be1.reshape(24, 1)
    be2c = be2.reshape(16, 1)
    bv1c = bv1.reshape(16, 1)
    bv2c = bv2.reshape(16, 1)
    bfc = bf.reshape(1, 1)

    HT = jnp.take(H, perm, axis=0).T
    Mcur = M
    for l in range(NUM_LAYER):
        MiT = jnp.take(Mcur, dst_s, axis=0).T
        MjT = jnp.take(Mcur, src_s, axis=0).T
        hT, mT = _mlp_layer(We1, be1c, We2, be2c, Wv1, bv1c, Wv2, bv2c,
                            MiT, MjT, HT)
        HT = hT
        if l < NUM_LAYER - 1:
            Magg = jax.ops.segment_max(mT.T, dst_s, num_segments=N_NODES)
            Mcur = jnp.where(jnp.isneginf(Magg), 0.0, Magg)

    o_sorted = _final_proj(Wf, bfc, HT)[0]
    inv = jnp.argsort(perm)
    out = jnp.take(o_sorted, inv)
    return out.reshape(E, 1)
